# Initial kernel scaffold; baseline (speedup 1.0000x reference)
import jax
import jax.numpy as jnp
from jax.experimental import pallas as pl


def kernel(movie_id, movie_categories, emb_movies, emb_cats, bias_movie, fc_w, fc_b):
    raise NotImplementedError("write your pallas kernel here")


# speedup vs baseline: 3.0701x; 3.0701x over previous
import functools

import jax
import jax.numpy as jnp
from jax import lax
from jax.experimental import pallas as pl
from jax.experimental.pallas import tpu as pltpu
from jax.experimental.pallas import tpu_sc as plsc

LANES = 16


def _sc_gather_kernel(B, L, nw, bpw, ncats, mdim):
  mesh = plsc.VectorSubcoreMesh(core_axis_name="c", subcore_axis_name="s")
  num_cores = mesh.num_cores

  @functools.partial(
      pl.kernel,
      out_type=(
          jax.ShapeDtypeStruct((B, mdim), jnp.float32),
          jax.ShapeDtypeStruct((B * 16,), jnp.float32),
          jax.ShapeDtypeStruct((B,), jnp.float32),
      ),
      mesh=mesh,
      compiler_params=pltpu.CompilerParams(
          needs_layout_passes=False, use_tc_tiling_on_sc=False),
      scratch_types=[
          pltpu.VMEM((bpw,), jnp.int32),
          pltpu.VMEM((bpw, mdim), jnp.float32),
          pltpu.VMEM((bpw,), jnp.int32),
          pltpu.VMEM((bpw, 16), jnp.float32),
          pltpu.VMEM((bpw,), jnp.float32),
          pltpu.VMEM((L, bpw), jnp.int32),
          pltpu.VMEM((ncats * 16,), jnp.float32),
          pltpu.VMEM((bpw * 16,), jnp.float32),
          pltpu.SemaphoreType.DMA,
          pltpu.SemaphoreType.DMA,
      ],
  )
  def body(mid_hbm, cats_hbm, movies_hbm, cattab_hbm, bias16_hbm,
           rows_out, mean_out, bias_out,
           idx_v, rows_v, idx16_v, b16_v, bias_v, cats_v, tab_v, mean_v,
           sem_r, sem_b):
    wid = lax.axis_index("s") * num_cores + lax.axis_index("c")
    base = wid * bpw
    pltpu.sync_copy(mid_hbm.at[pl.ds(base, bpw)], idx_v)
    cp_rows = pltpu.async_copy(movies_hbm.at[idx_v], rows_v, sem_r)

    def shift_grp(g, carry):
      b0 = g * LANES
      idx16_v[pl.ds(b0, LANES)] = lax.shift_right_logical(
          idx_v[pl.ds(b0, LANES)], 4)
      return carry
    lax.fori_loop(0, bpw // LANES, shift_grp, 0)
    cp_bias = pltpu.async_copy(bias16_hbm.at[idx16_v], b16_v, sem_b)

    pltpu.sync_copy(cats_hbm.at[:, pl.ds(base, bpw)], cats_v)
    pltpu.sync_copy(cattab_hbm, tab_v)

    lane = lax.iota(jnp.int32, LANES)

    def group(g, carry):
      b0 = g * LANES
      cnt = jnp.zeros((LANES,), jnp.float32)
      acc = [jnp.zeros((LANES,), jnp.float32) for _ in range(16)]
      for l in range(L):
        idxs = cats_v[l, pl.ds(b0, LANES)]
        cnt = cnt + jnp.where(idxs != 0, 1.0, 0.0)
        flat = idxs * 16
        for d in range(16):
          acc[d] = acc[d] + plsc.load_gather(tab_v, [flat + d])
      inv = jnp.where(cnt > 0, 1.0 / jnp.maximum(cnt, 1.0), 0.0)
      pos0 = (b0 + lane) * 16
      for d in range(16):
        plsc.store_scatter(mean_v, [pos0 + d], acc[d] * inv)
      return carry

    lax.fori_loop(0, bpw // LANES, group, 0)

    pltpu.sync_copy(mean_v, mean_out.at[pl.ds(base * 16, bpw * 16)])
    cp_rows.wait()
    pltpu.sync_copy(rows_v, rows_out.at[pl.ds(base, bpw), :])

    cp_bias.wait()
    def bias_grp(g, carry):
      b0 = g * LANES
      idxs = idx_v[pl.ds(b0, LANES)]
      off = jnp.bitwise_and(idxs, 15)
      bias_v[pl.ds(b0, LANES)] = plsc.load_gather(b16_v, [b0 + lane, off])
      return carry
    lax.fori_loop(0, bpw // LANES, bias_grp, 0)
    pltpu.sync_copy(bias_v, bias_out.at[pl.ds(base, bpw)])

  return body


def _fc_body(rows_ref, mean_ref, w1_ref, w2_ref, b_ref, out_ref):
  a = jnp.maximum(rows_ref[...], 0.0)
  c = jnp.maximum(mean_ref[...], 0.0)
  out_ref[...] = (
      jnp.dot(a, w1_ref[...], preferred_element_type=jnp.float32)
      + jnp.dot(c, w2_ref[...], preferred_element_type=jnp.float32)
      + b_ref[...]
  )


def kernel(movie_id, movie_categories, emb_movies, emb_cats, bias_movie,
           fc_w, fc_b):
  B = movie_id.shape[0]
  L = movie_categories.shape[1]
  ncats, cdim = emb_cats.shape
  mdim = emb_movies.shape[1]
  assert cdim == 16

  info = plsc.get_sparse_core_info()
  nw = info.num_cores * info.num_subcores
  bpw = B // nw

  mid = movie_id.astype(jnp.int32)
  cats_t = movie_categories.astype(jnp.int32).T
  tab_flat = emb_cats.reshape(-1)

  bias16 = bias_movie.reshape(-1, 16)
  sc = _sc_gather_kernel(B, L, nw, bpw, ncats, mdim)
  rows, mean_flat, bias = sc(mid, cats_t, emb_movies, tab_flat, bias16)
  mean = mean_flat.reshape(B, cdim)

  w1 = fc_w.T[:mdim]
  w2 = fc_w.T[mdim:]
  out_dim = fc_w.shape[0]

  movie_vec = pl.pallas_call(
      _fc_body,
      out_shape=jax.ShapeDtypeStruct((B, out_dim), jnp.float32),
  )(rows, mean, w1, w2, fc_b.reshape(1, out_dim))

  return movie_vec, bias
